# baseline (speedup 1.0000x reference)
import jax
import jax.numpy as jnp
from jax import lax
from jax.experimental import pallas as pl
from jax.experimental.pallas import tpu as pltpu
from jax.experimental.pallas import tpu_sc as plsc

N = 10000
D = 128
H = 64
C = 16
K = 10
ALPHA = 0.1

NC = 2
NSC = 16
NWK = NC * NSC
CH = 512
NCH = 20
NB = NCH
EW = NCH * CH
EP = NWK * EW
NJUNK = 16
NP = 10240
NH = NP // NC
NR = NP // NSC
NSYNC = 2 * K + 4
MAGIC = 0x5CA1AB1E

_RSQ = 0x5F3759DF


def _mlp_body(x_ref, w1_ref, b1_ref, w2_ref, b2_ref, o_ref):
    h1 = lax.dot_general(x_ref[...], w1_ref[...], (((1,), (1,)), ((), ())),
                         preferred_element_type=jnp.float32)
    h1 = jnp.maximum(h1 + b1_ref[...], 0.0)
    h2 = lax.dot_general(h1, w2_ref[...], (((1,), (1,)), ((), ())),
                         preferred_element_type=jnp.float32)
    o_ref[...] = jnp.maximum(h2 + b2_ref[...], 0.0)


_mlp = pl.pallas_call(
    _mlp_body,
    grid=(10,),
    in_specs=[
        pl.BlockSpec((N // 10, D), lambda i: (i, 0)),
        pl.BlockSpec((H, D), lambda i: (0, 0)),
        pl.BlockSpec((1, H), lambda i: (0, 0)),
        pl.BlockSpec((C, H), lambda i: (0, 0)),
        pl.BlockSpec((1, C), lambda i: (0, 0)),
    ],
    out_specs=pl.BlockSpec((N // 10, C), lambda i: (i, 0)),
    out_shape=jax.ShapeDtypeStruct((N, C), jnp.float32),
)


def _frsqrt(x):
    i = lax.bitcast_convert_type(x, jnp.int32)
    y = lax.bitcast_convert_type(_RSQ - (i >> 1), jnp.float32)
    for _ in range(3):
        y = y * (1.5 - 0.5 * x * y * y)
    return y


def _prop_body(h_hbm, src_hbm, dst_hbm,
               out_hbm, xacc_hbm,
               t_sh, acc_sh,
               src_t, dst_t, msg_a, msg_b,
               ab, bb, zbuf, tl, d2l, gl, dvl,
               gs_a, gs_b, ss_a, ss_b, xsem):
    cid = lax.axis_index("c")
    sid = lax.axis_index("s")
    wid = cid * NSC + sid
    oid = 1 - cid
    nbase = sid * NR

    zero16 = jnp.zeros((16,), jnp.float32)
    one16 = jnp.ones((16,), jnp.float32)

    def exchange(slot):
        plsc.subcore_barrier()
        pltpu.sync_copy(acc_sh.at[pl.ds(nbase, NR)],
                        xacc_hbm.at[cid].at[slot].at[pl.ds(nbase, NR)])
        pl.semaphore_signal(xsem, 1, core_index=oid)
        pl.semaphore_wait(xsem, 1)
        pltpu.sync_copy(xacc_hbm.at[oid].at[slot].at[pl.ds(nbase, NR)], bb)

    pltpu.sync_copy(src_hbm.at[wid], src_t)
    pltpu.sync_copy(dst_hbm.at[wid], dst_t)

    @pl.loop(0, NR)
    def _(i):
        zbuf[i] = zero16

    pltpu.sync_copy(zbuf, acc_sh.at[pl.ds(nbase, NR)])
    plsc.subcore_barrier()

    @pl.loop(0, CH)
    def _(i):
        msg_a[0, i] = one16

    @pl.loop(0, NCH)
    def _(c):
        pltpu.sync_copy(msg_a.at[0], acc_sh.at[dst_t.at[c]], add=True)

    exchange(0)

    pltpu.sync_copy(h_hbm.at[pl.ds(nbase, NR)], tl)
    pltpu.sync_copy(acc_sh.at[pl.ds(nbase, NR)], ab)
    pltpu.sync_copy(zbuf, acc_sh.at[pl.ds(nbase, NR)])

    @pl.loop(0, NR, unroll=8)
    def _(i):
        deg = ab[i] + bb[i] + 1.0
        dv = _frsqrt(deg)
        t0 = dv * tl[i]
        tl[i] = t0
        gl[i] = ALPHA * t0
        d2l[i] = (1.0 - ALPHA) * dv * dv
        dvl[i] = dv

    pltpu.sync_copy(tl, t_sh.at[pl.ds(nbase, NR)])
    plsc.subcore_barrier()

    def g_start(c, buf, sem):
        pltpu.async_copy(t_sh.at[src_t.at[c]], buf.at[0], sem)

    def g_wait(c, buf, sem):
        pltpu.make_async_copy(t_sh.at[src_t.at[c]], buf.at[0], sem).wait()

    def s_start(c, buf, sem):
        pltpu.async_copy(buf.at[0], acc_sh.at[dst_t.at[c]], sem, add=True)

    def s_wait(c, buf, sem):
        pltpu.make_async_copy(buf.at[0], acc_sh.at[dst_t.at[c]], sem).wait()

    @pl.loop(0, K)
    def _(k):
        g_start(0, msg_a, gs_a)
        g_wait(0, msg_a, gs_a)
        g_start(1, msg_b, gs_b)
        s_start(0, msg_a, ss_a)

        @pl.loop(1, NB // 2)
        def _(p):
            b0 = 2 * p
            b1 = b0 + 1
            s_wait(b0 - 2, msg_a, ss_a)
            g_start(b0, msg_a, gs_a)
            g_wait(b0 - 1, msg_b, gs_b)
            s_start(b0 - 1, msg_b, ss_b)
            s_wait(b0 - 1, msg_b, ss_b)
            g_start(b1, msg_b, gs_b)
            g_wait(b0, msg_a, gs_a)
            s_start(b0, msg_a, ss_a)

        g_wait(NB - 1, msg_b, gs_b)
        s_wait(NB - 2, msg_a, ss_a)
        s_start(NB - 1, msg_b, ss_b)
        s_wait(NB - 1, msg_b, ss_b)

        exchange((k + 1) % 2)

        pltpu.sync_copy(acc_sh.at[pl.ds(nbase, NR)], ab)
        pltpu.sync_copy(zbuf, acc_sh.at[pl.ds(nbase, NR)])

        @pl.loop(0, NR, unroll=8)
        def _(i):
            tl[i] = d2l[i] * (ab[i] + bb[i] + tl[i]) + gl[i]

        pltpu.sync_copy(tl, t_sh.at[pl.ds(nbase, NR)])
        plsc.subcore_barrier()

    @pl.loop(0, NR, unroll=8)
    def _(i):
        bb[i] = tl[i] / dvl[i]

    pltpu.sync_copy(bb, out_hbm.at[pl.ds(nbase, NR)])


_prop = pl.kernel(
    _prop_body,
    out_type=(
        jax.ShapeDtypeStruct((NP, C), jnp.float32),
        jax.ShapeDtypeStruct((NC, 2, NP, C), jnp.float32),
    ),
    mesh=plsc.VectorSubcoreMesh(core_axis_name="c", subcore_axis_name="s",
                                num_cores=NC, num_subcores=NSC),
    compiler_params=pltpu.CompilerParams(use_tc_tiling_on_sc=False),
    scratch_types=[
        pltpu.VMEM_SHARED((NP, C), jnp.float32),
        pltpu.VMEM_SHARED((NP, C), jnp.float32),
        pltpu.VMEM((NCH, CH), jnp.int32),
        pltpu.VMEM((NCH, CH), jnp.int32),
        pltpu.VMEM((1, CH, C), jnp.float32),
        pltpu.VMEM((1, CH, C), jnp.float32),
        pltpu.VMEM((NR, C), jnp.float32),
        pltpu.VMEM((NR, C), jnp.float32),
        pltpu.VMEM((NR, C), jnp.float32),
        pltpu.VMEM((NR, C), jnp.float32),
        pltpu.VMEM((NR, C), jnp.float32),
        pltpu.VMEM((NR, C), jnp.float32),
        pltpu.VMEM((NR, C), jnp.float32),
        pltpu.SemaphoreType.DMA,
        pltpu.SemaphoreType.DMA,
        pltpu.SemaphoreType.DMA,
        pltpu.SemaphoreType.DMA,
        pltpu.SemaphoreType.REGULAR,
    ],
)


@jax.jit
def kernel(x, edge_index, W1, b1, W2, b2):
    h = _mlp(x, W1, b1.reshape(1, H), W2, b2.reshape(1, C))

    npad = EP - edge_index.shape[1]
    rng = jnp.arange(npad, dtype=jnp.int32)
    src = jnp.concatenate([edge_index[0], rng % 64])
    dst = jnp.concatenate([edge_index[1], N + (rng % NJUNK)])
    src_r = src.reshape(NWK, NCH, CH)
    dst_r = dst.reshape(NWK, NCH, CH)

    h_pad = jnp.concatenate([h, jnp.zeros((NP - N, C), jnp.float32)])
    return _prop(h_pad, src_r, dst_r)[0][:N]


# speedup vs baseline: 1.1936x; 1.1936x over previous
import jax
import jax.numpy as jnp
from jax import lax
from jax.experimental import pallas as pl
from jax.experimental.pallas import tpu as pltpu
from jax.experimental.pallas import tpu_sc as plsc

N = 10000
D = 128
H = 64
C = 16
K = 10
ALPHA = 0.1

NC = 2
NSC = 16
NWK = NC * NSC
CH = 512
NCH = 20
NB = NCH
EW = NCH * CH
EP = NWK * EW
NJUNK = 16
NP = 10240
NH = NP // NC
NR = NP // NWK
NSYNC = 2 * K + 4
MAGIC = 0x5CA1AB1E

_RSQ = 0x5F3759DF


def _mlp_body(x_ref, w1_ref, b1_ref, w2_ref, b2_ref, o_ref):
    h1 = lax.dot_general(x_ref[...], w1_ref[...], (((1,), (1,)), ((), ())),
                         preferred_element_type=jnp.float32)
    h1 = jnp.maximum(h1 + b1_ref[...], 0.0)
    h2 = lax.dot_general(h1, w2_ref[...], (((1,), (1,)), ((), ())),
                         preferred_element_type=jnp.float32)
    o_ref[...] = jnp.maximum(h2 + b2_ref[...], 0.0)


_mlp = pl.pallas_call(
    _mlp_body,
    grid=(10,),
    in_specs=[
        pl.BlockSpec((N // 10, D), lambda i: (i, 0)),
        pl.BlockSpec((H, D), lambda i: (0, 0)),
        pl.BlockSpec((1, H), lambda i: (0, 0)),
        pl.BlockSpec((C, H), lambda i: (0, 0)),
        pl.BlockSpec((1, C), lambda i: (0, 0)),
    ],
    out_specs=pl.BlockSpec((N // 10, C), lambda i: (i, 0)),
    out_shape=jax.ShapeDtypeStruct((N, C), jnp.float32),
)


def _frsqrt(x):
    i = lax.bitcast_convert_type(x, jnp.int32)
    y = lax.bitcast_convert_type(_RSQ - (i >> 1), jnp.float32)
    for _ in range(3):
        y = y * (1.5 - 0.5 * x * y * y)
    return y


def _prop_body(h_hbm, src_hbm, dst_hbm,
               out_hbm, xacc_hbm, xt_hbm,
               t_sh, acc_sh,
               src_t, dst_t, msg_a, msg_b,
               ab, bb, zbuf, tl, d2l, gl, dvl,
               gs_a, gs_b, ss_a, ss_b, xsem):
    cid = lax.axis_index("c")
    sid = lax.axis_index("s")
    wid = cid * NSC + sid
    oid = 1 - cid
    nbase = wid * NR
    loff = sid * NR
    obase = oid * NH + loff

    zero16 = jnp.zeros((16,), jnp.float32)
    one16 = jnp.ones((16,), jnp.float32)

    def cross_sync():
        plsc.subcore_barrier()
        pl.semaphore_signal(xsem, 1, core_index=oid)
        pl.semaphore_wait(xsem, 1)

    pltpu.sync_copy(src_hbm.at[wid], src_t)
    pltpu.sync_copy(dst_hbm.at[wid], dst_t)

    @pl.loop(0, NR)
    def _(i):
        zbuf[i] = zero16

    pltpu.sync_copy(zbuf, acc_sh.at[pl.ds(cid * NH + loff, NR)])
    pltpu.sync_copy(zbuf, acc_sh.at[pl.ds(obase, NR)])
    plsc.subcore_barrier()

    @pl.loop(0, CH)
    def _(i):
        msg_a[0, i] = one16

    @pl.loop(0, NCH)
    def _(c):
        pltpu.sync_copy(msg_a.at[0], acc_sh.at[dst_t.at[c]], add=True)
    plsc.subcore_barrier()

    pltpu.sync_copy(acc_sh.at[pl.ds(obase, NR)],
                    xacc_hbm.at[cid].at[pl.ds(loff, NR)])
    pltpu.sync_copy(zbuf, acc_sh.at[pl.ds(obase, NR)])
    cross_sync()

    pltpu.sync_copy(h_hbm.at[pl.ds(nbase, NR)], tl)
    pltpu.sync_copy(acc_sh.at[pl.ds(nbase, NR)], ab)
    pltpu.sync_copy(zbuf, acc_sh.at[pl.ds(nbase, NR)])
    pltpu.sync_copy(xacc_hbm.at[oid].at[pl.ds(loff, NR)], bb)

    @pl.loop(0, NR)
    def _(i):
        deg = ab[i] + bb[i] + 1.0
        dv = _frsqrt(deg)
        t0 = dv * tl[i]
        tl[i] = t0
        gl[i] = ALPHA * t0
        d2l[i] = (1.0 - ALPHA) * dv * dv
        dvl[i] = dv

    pltpu.sync_copy(tl, t_sh.at[pl.ds(nbase, NR)])
    pltpu.sync_copy(tl, xt_hbm.at[cid].at[pl.ds(loff, NR)])
    cross_sync()
    pltpu.sync_copy(xt_hbm.at[oid].at[pl.ds(loff, NR)],
                    t_sh.at[pl.ds(obase, NR)])
    plsc.subcore_barrier()

    def g_start(c, buf, sem):
        pltpu.async_copy(t_sh.at[src_t.at[c]], buf.at[0], sem)

    def g_wait(c, buf, sem):
        pltpu.make_async_copy(t_sh.at[src_t.at[c]], buf.at[0], sem).wait()

    def s_start(c, buf, sem):
        pltpu.async_copy(buf.at[0], acc_sh.at[dst_t.at[c]], sem, add=True)

    def s_wait(c, buf, sem):
        pltpu.make_async_copy(buf.at[0], acc_sh.at[dst_t.at[c]], sem).wait()

    @pl.loop(0, K)
    def _(k):
        g_start(0, msg_a, gs_a)
        g_wait(0, msg_a, gs_a)
        g_start(1, msg_b, gs_b)
        s_start(0, msg_a, ss_a)

        @pl.loop(1, NB // 2)
        def _(p):
            b0 = 2 * p
            b1 = b0 + 1
            s_wait(b0 - 2, msg_a, ss_a)
            g_start(b0, msg_a, gs_a)
            g_wait(b0 - 1, msg_b, gs_b)
            s_start(b0 - 1, msg_b, ss_b)
            s_wait(b0 - 1, msg_b, ss_b)
            g_start(b1, msg_b, gs_b)
            g_wait(b0, msg_a, gs_a)
            s_start(b0, msg_a, ss_a)

        g_wait(NB - 1, msg_b, gs_b)
        s_wait(NB - 2, msg_a, ss_a)
        s_start(NB - 1, msg_b, ss_b)
        s_wait(NB - 1, msg_b, ss_b)
        plsc.subcore_barrier()

        pltpu.sync_copy(acc_sh.at[pl.ds(obase, NR)],
                        xacc_hbm.at[cid].at[pl.ds(loff, NR)])
        pltpu.async_copy(zbuf, acc_sh.at[pl.ds(obase, NR)], ss_b)
        pl.semaphore_signal(xsem, 1, core_index=oid)
        pltpu.make_async_copy(zbuf, acc_sh.at[pl.ds(obase, NR)], ss_b).wait()
        pl.semaphore_wait(xsem, 1)

        pltpu.async_copy(acc_sh.at[pl.ds(nbase, NR)], ab, gs_a)
        pltpu.async_copy(xacc_hbm.at[oid].at[pl.ds(loff, NR)], bb, gs_b)
        pltpu.make_async_copy(acc_sh.at[pl.ds(nbase, NR)], ab, gs_a).wait()
        pltpu.async_copy(zbuf, acc_sh.at[pl.ds(nbase, NR)], ss_a)
        pltpu.make_async_copy(xacc_hbm.at[oid].at[pl.ds(loff, NR)], bb,
                              gs_b).wait()

        @pl.loop(0, NR)
        def _(i):
            tl[i] = d2l[i] * (ab[i] + bb[i] + tl[i]) + gl[i]

        pltpu.async_copy(tl, t_sh.at[pl.ds(nbase, NR)], ss_b)
        pltpu.sync_copy(tl, xt_hbm.at[cid].at[pl.ds(loff, NR)])
        pltpu.make_async_copy(zbuf, acc_sh.at[pl.ds(nbase, NR)], ss_a).wait()
        pltpu.make_async_copy(tl, t_sh.at[pl.ds(nbase, NR)], ss_b).wait()
        cross_sync()
        pltpu.sync_copy(xt_hbm.at[oid].at[pl.ds(loff, NR)],
                        t_sh.at[pl.ds(obase, NR)])
        plsc.subcore_barrier()

    @pl.loop(0, NR)
    def _(i):
        bb[i] = tl[i] / dvl[i]

    pltpu.sync_copy(bb, out_hbm.at[pl.ds(nbase, NR)])


_prop = pl.kernel(
    _prop_body,
    out_type=(
        jax.ShapeDtypeStruct((NP, C), jnp.float32),
        jax.ShapeDtypeStruct((NC, NH, C), jnp.float32),
        jax.ShapeDtypeStruct((NC, NH, C), jnp.float32),
    ),
    mesh=plsc.VectorSubcoreMesh(core_axis_name="c", subcore_axis_name="s",
                                num_cores=NC, num_subcores=NSC),
    compiler_params=pltpu.CompilerParams(use_tc_tiling_on_sc=False),
    scratch_types=[
        pltpu.VMEM_SHARED((NP, C), jnp.float32),
        pltpu.VMEM_SHARED((NP, C), jnp.float32),
        pltpu.VMEM((NCH, CH), jnp.int32),
        pltpu.VMEM((NCH, CH), jnp.int32),
        pltpu.VMEM((1, CH, C), jnp.float32),
        pltpu.VMEM((1, CH, C), jnp.float32),
        pltpu.VMEM((NR, C), jnp.float32),
        pltpu.VMEM((NR, C), jnp.float32),
        pltpu.VMEM((NR, C), jnp.float32),
        pltpu.VMEM((NR, C), jnp.float32),
        pltpu.VMEM((NR, C), jnp.float32),
        pltpu.VMEM((NR, C), jnp.float32),
        pltpu.VMEM((NR, C), jnp.float32),
        pltpu.SemaphoreType.DMA,
        pltpu.SemaphoreType.DMA,
        pltpu.SemaphoreType.DMA,
        pltpu.SemaphoreType.DMA,
        pltpu.SemaphoreType.REGULAR,
    ],
)


@jax.jit
def kernel(x, edge_index, W1, b1, W2, b2):
    h = _mlp(x, W1, b1.reshape(1, H), W2, b2.reshape(1, C))

    npad = EP - edge_index.shape[1]
    rng = jnp.arange(npad, dtype=jnp.int32)
    src = jnp.concatenate([edge_index[0], rng % 64])
    dst = jnp.concatenate([edge_index[1], N + (rng % NJUNK)])
    src_r = src.reshape(NWK, NCH, CH)
    dst_r = dst.reshape(NWK, NCH, CH)

    h_pad = jnp.concatenate([h, jnp.zeros((NP - N, C), jnp.float32)])
    return _prop(h_pad, src_r, dst_r)[0][:N]


# speedup vs baseline: 1.2617x; 1.0570x over previous
import jax
import jax.numpy as jnp
from jax import lax
from jax.experimental import pallas as pl
from jax.experimental.pallas import tpu as pltpu
from jax.experimental.pallas import tpu_sc as plsc

N = 10000
D = 128
H = 64
C = 16
K = 10
ALPHA = 0.1

NC = 2
NSC = 16
NWK = NC * NSC
CH = 1024
NCH = 10
NB = NCH
EW = NCH * CH
EP = NWK * EW
NJUNK = 16
NP = 10240
NH = NP // NC
NR = NP // NWK
NSYNC = 2 * K + 4
MAGIC = 0x5CA1AB1E

_RSQ = 0x5F3759DF


def _mlp_body(x_ref, w1_ref, b1_ref, w2_ref, b2_ref, o_ref):
    h1 = lax.dot_general(x_ref[...], w1_ref[...], (((1,), (1,)), ((), ())),
                         preferred_element_type=jnp.float32)
    h1 = jnp.maximum(h1 + b1_ref[...], 0.0)
    h2 = lax.dot_general(h1, w2_ref[...], (((1,), (1,)), ((), ())),
                         preferred_element_type=jnp.float32)
    o_ref[...] = jnp.maximum(h2 + b2_ref[...], 0.0)


_mlp = pl.pallas_call(
    _mlp_body,
    grid=(10,),
    in_specs=[
        pl.BlockSpec((N // 10, D), lambda i: (i, 0)),
        pl.BlockSpec((H, D), lambda i: (0, 0)),
        pl.BlockSpec((1, H), lambda i: (0, 0)),
        pl.BlockSpec((C, H), lambda i: (0, 0)),
        pl.BlockSpec((1, C), lambda i: (0, 0)),
    ],
    out_specs=pl.BlockSpec((N // 10, C), lambda i: (i, 0)),
    out_shape=jax.ShapeDtypeStruct((N, C), jnp.float32),
)


def _frsqrt(x):
    i = lax.bitcast_convert_type(x, jnp.int32)
    y = lax.bitcast_convert_type(_RSQ - (i >> 1), jnp.float32)
    for _ in range(3):
        y = y * (1.5 - 0.5 * x * y * y)
    return y


def _prop_body(h_hbm, src_hbm, dst_hbm,
               out_hbm, xacc_hbm, xt_hbm,
               t_sh, acc_sh,
               src_t, dst_t, msg_a, msg_b,
               ab, bb, zbuf, tl, d2l, gl, dvl,
               gs_a, gs_b, ss_a, ss_b, xsem):
    cid = lax.axis_index("c")
    sid = lax.axis_index("s")
    wid = cid * NSC + sid
    oid = 1 - cid
    nbase = wid * NR
    loff = sid * NR
    obase = oid * NH + loff

    zero16 = jnp.zeros((16,), jnp.float32)
    one16 = jnp.ones((16,), jnp.float32)

    def cross_sync():
        plsc.subcore_barrier()
        pl.semaphore_signal(xsem, 1, core_index=oid)
        pl.semaphore_wait(xsem, 1)

    pltpu.sync_copy(src_hbm.at[wid], src_t)
    pltpu.sync_copy(dst_hbm.at[wid], dst_t)

    @pl.loop(0, NR)
    def _(i):
        zbuf[i] = zero16

    pltpu.sync_copy(zbuf, acc_sh.at[pl.ds(cid * NH + loff, NR)])
    pltpu.sync_copy(zbuf, acc_sh.at[pl.ds(obase, NR)])
    plsc.subcore_barrier()

    @pl.loop(0, CH)
    def _(i):
        msg_a[0, i] = one16

    @pl.loop(0, NCH)
    def _(c):
        pltpu.sync_copy(msg_a.at[0], acc_sh.at[dst_t.at[c]], add=True)
    plsc.subcore_barrier()

    pltpu.sync_copy(acc_sh.at[pl.ds(obase, NR)],
                    xacc_hbm.at[cid].at[pl.ds(loff, NR)])
    pltpu.sync_copy(zbuf, acc_sh.at[pl.ds(obase, NR)])
    cross_sync()

    pltpu.sync_copy(h_hbm.at[pl.ds(nbase, NR)], tl)
    pltpu.sync_copy(acc_sh.at[pl.ds(nbase, NR)], ab)
    pltpu.sync_copy(zbuf, acc_sh.at[pl.ds(nbase, NR)])
    pltpu.sync_copy(xacc_hbm.at[oid].at[pl.ds(loff, NR)], bb)

    @pl.loop(0, NR)
    def _(i):
        deg = ab[i] + bb[i] + 1.0
        dv = _frsqrt(deg)
        t0 = dv * tl[i]
        tl[i] = t0
        gl[i] = ALPHA * t0
        d2l[i] = (1.0 - ALPHA) * dv * dv
        dvl[i] = dv

    pltpu.sync_copy(tl, t_sh.at[pl.ds(nbase, NR)])
    pltpu.sync_copy(tl, xt_hbm.at[cid].at[pl.ds(loff, NR)])
    cross_sync()
    pltpu.sync_copy(xt_hbm.at[oid].at[pl.ds(loff, NR)],
                    t_sh.at[pl.ds(obase, NR)])
    plsc.subcore_barrier()

    def g_start(c, buf, sem):
        pltpu.async_copy(t_sh.at[src_t.at[c]], buf.at[0], sem)

    def g_wait(c, buf, sem):
        pltpu.make_async_copy(t_sh.at[src_t.at[c]], buf.at[0], sem).wait()

    def s_start(c, buf, sem):
        pltpu.async_copy(buf.at[0], acc_sh.at[dst_t.at[c]], sem, add=True)

    def s_wait(c, buf, sem):
        pltpu.make_async_copy(buf.at[0], acc_sh.at[dst_t.at[c]], sem).wait()

    @pl.loop(0, K)
    def _(k):
        g_start(0, msg_a, gs_a)
        g_wait(0, msg_a, gs_a)
        g_start(1, msg_b, gs_b)
        s_start(0, msg_a, ss_a)

        @pl.loop(1, NB // 2)
        def _(p):
            b0 = 2 * p
            b1 = b0 + 1
            s_wait(b0 - 2, msg_a, ss_a)
            g_start(b0, msg_a, gs_a)
            g_wait(b0 - 1, msg_b, gs_b)
            s_start(b0 - 1, msg_b, ss_b)
            s_wait(b0 - 1, msg_b, ss_b)
            g_start(b1, msg_b, gs_b)
            g_wait(b0, msg_a, gs_a)
            s_start(b0, msg_a, ss_a)

        g_wait(NB - 1, msg_b, gs_b)
        s_wait(NB - 2, msg_a, ss_a)
        s_start(NB - 1, msg_b, ss_b)
        s_wait(NB - 1, msg_b, ss_b)
        plsc.subcore_barrier()

        pltpu.sync_copy(acc_sh.at[pl.ds(obase, NR)],
                        xacc_hbm.at[cid].at[pl.ds(loff, NR)])
        pltpu.async_copy(zbuf, acc_sh.at[pl.ds(obase, NR)], ss_b)
        pl.semaphore_signal(xsem, 1, core_index=oid)
        pltpu.make_async_copy(zbuf, acc_sh.at[pl.ds(obase, NR)], ss_b).wait()
        pl.semaphore_wait(xsem, 1)

        pltpu.async_copy(acc_sh.at[pl.ds(nbase, NR)], ab, gs_a)
        pltpu.async_copy(xacc_hbm.at[oid].at[pl.ds(loff, NR)], bb, gs_b)
        pltpu.make_async_copy(acc_sh.at[pl.ds(nbase, NR)], ab, gs_a).wait()
        pltpu.async_copy(zbuf, acc_sh.at[pl.ds(nbase, NR)], ss_a)
        pltpu.make_async_copy(xacc_hbm.at[oid].at[pl.ds(loff, NR)], bb,
                              gs_b).wait()

        @pl.loop(0, NR)
        def _(i):
            tl[i] = d2l[i] * (ab[i] + bb[i] + tl[i]) + gl[i]

        pltpu.async_copy(tl, t_sh.at[pl.ds(nbase, NR)], ss_b)
        pltpu.sync_copy(tl, xt_hbm.at[cid].at[pl.ds(loff, NR)])
        pltpu.make_async_copy(zbuf, acc_sh.at[pl.ds(nbase, NR)], ss_a).wait()
        pltpu.make_async_copy(tl, t_sh.at[pl.ds(nbase, NR)], ss_b).wait()
        cross_sync()
        pltpu.sync_copy(xt_hbm.at[oid].at[pl.ds(loff, NR)],
                        t_sh.at[pl.ds(obase, NR)])
        plsc.subcore_barrier()

    @pl.loop(0, NR)
    def _(i):
        bb[i] = tl[i] / dvl[i]

    pltpu.sync_copy(bb, out_hbm.at[pl.ds(nbase, NR)])


_prop = pl.kernel(
    _prop_body,
    out_type=(
        jax.ShapeDtypeStruct((NP, C), jnp.float32),
        jax.ShapeDtypeStruct((NC, NH, C), jnp.float32),
        jax.ShapeDtypeStruct((NC, NH, C), jnp.float32),
    ),
    mesh=plsc.VectorSubcoreMesh(core_axis_name="c", subcore_axis_name="s",
                                num_cores=NC, num_subcores=NSC),
    compiler_params=pltpu.CompilerParams(use_tc_tiling_on_sc=False),
    scratch_types=[
        pltpu.VMEM_SHARED((NP, C), jnp.float32),
        pltpu.VMEM_SHARED((NP, C), jnp.float32),
        pltpu.VMEM((NCH, CH), jnp.int32),
        pltpu.VMEM((NCH, CH), jnp.int32),
        pltpu.VMEM((1, CH, C), jnp.float32),
        pltpu.VMEM((1, CH, C), jnp.float32),
        pltpu.VMEM((NR, C), jnp.float32),
        pltpu.VMEM((NR, C), jnp.float32),
        pltpu.VMEM((NR, C), jnp.float32),
        pltpu.VMEM((NR, C), jnp.float32),
        pltpu.VMEM((NR, C), jnp.float32),
        pltpu.VMEM((NR, C), jnp.float32),
        pltpu.VMEM((NR, C), jnp.float32),
        pltpu.SemaphoreType.DMA,
        pltpu.SemaphoreType.DMA,
        pltpu.SemaphoreType.DMA,
        pltpu.SemaphoreType.DMA,
        pltpu.SemaphoreType.REGULAR,
    ],
)


@jax.jit
def kernel(x, edge_index, W1, b1, W2, b2):
    h = _mlp(x, W1, b1.reshape(1, H), W2, b2.reshape(1, C))

    npad = EP - edge_index.shape[1]
    rng = jnp.arange(npad, dtype=jnp.int32)
    src = jnp.concatenate([edge_index[0], rng % 64])
    dst = jnp.concatenate([edge_index[1], N + (rng % NJUNK)])
    src_r = src.reshape(NWK, NCH, CH)
    dst_r = dst.reshape(NWK, NCH, CH)

    h_pad = jnp.concatenate([h, jnp.zeros((NP - N, C), jnp.float32)])
    return _prop(h_pad, src_r, dst_r)[0][:N]


# speedup vs baseline: 1.2621x; 1.0003x over previous
import jax
import jax.numpy as jnp
from jax import lax
from jax.experimental import pallas as pl
from jax.experimental.pallas import tpu as pltpu
from jax.experimental.pallas import tpu_sc as plsc

N = 10000
D = 128
H = 64
C = 16
K = 10
ALPHA = 0.1

NC = 2
NSC = 16
NWK = NC * NSC
CH = 1024
NCH = 10
NB = NCH
EW = NCH * CH
EP = NWK * EW
NJUNK = 16
NP = 10240
NH = NP // NC
NR = NP // NWK
_RSQ = 0x5F3759DF


def _mlp_body(x_ref, w1_ref, b1_ref, w2_ref, b2_ref, o_ref):
    h1 = lax.dot_general(x_ref[...], w1_ref[...], (((1,), (1,)), ((), ())),
                         preferred_element_type=jnp.float32)
    h1 = jnp.maximum(h1 + b1_ref[...], 0.0)
    h2 = lax.dot_general(h1, w2_ref[...], (((1,), (1,)), ((), ())),
                         preferred_element_type=jnp.float32)
    o_ref[...] = jnp.maximum(h2 + b2_ref[...], 0.0)


_mlp = pl.pallas_call(
    _mlp_body,
    grid=(10,),
    in_specs=[
        pl.BlockSpec((N // 10, D), lambda i: (i, 0)),
        pl.BlockSpec((H, D), lambda i: (0, 0)),
        pl.BlockSpec((1, H), lambda i: (0, 0)),
        pl.BlockSpec((C, H), lambda i: (0, 0)),
        pl.BlockSpec((1, C), lambda i: (0, 0)),
    ],
    out_specs=pl.BlockSpec((N // 10, C), lambda i: (i, 0)),
    out_shape=jax.ShapeDtypeStruct((N, C), jnp.float32),
)


def _frsqrt(x):
    i = lax.bitcast_convert_type(x, jnp.int32)
    y = lax.bitcast_convert_type(_RSQ - (i >> 1), jnp.float32)
    for _ in range(3):
        y = y * (1.5 - 0.5 * x * y * y)
    return y


def _prop_body(h_hbm, src_hbm, dst_hbm,
               out_hbm, xacc_hbm, xt_hbm,
               t_sh, acc_sh,
               src_t, dst_t, msg_a, msg_b,
               ab, bb, zbuf, tl, d2l, gl, dvl,
               gs_a, gs_b, ss_a, ss_b, xsem):
    cid = lax.axis_index("c")
    sid = lax.axis_index("s")
    wid = cid * NSC + sid
    oid = 1 - cid
    nbase = wid * NR
    loff = sid * NR
    obase = oid * NH + loff

    zero16 = jnp.zeros((16,), jnp.float32)
    one16 = jnp.ones((16,), jnp.float32)

    def cross_sync():
        plsc.subcore_barrier()
        pl.semaphore_signal(xsem, 1, core_index=oid)
        pl.semaphore_wait(xsem, 1)

    pltpu.sync_copy(src_hbm.at[wid], src_t)
    pltpu.sync_copy(dst_hbm.at[wid], dst_t)

    @pl.loop(0, NR)
    def _(i):
        zbuf[i] = zero16

    pltpu.sync_copy(zbuf, acc_sh.at[pl.ds(cid * NH + loff, NR)])
    pltpu.sync_copy(zbuf, acc_sh.at[pl.ds(obase, NR)])
    plsc.subcore_barrier()

    @pl.loop(0, CH)
    def _(i):
        msg_a[0, i] = one16

    @pl.loop(0, NCH)
    def _(c):
        pltpu.sync_copy(msg_a.at[0], acc_sh.at[dst_t.at[c]], add=True)
    plsc.subcore_barrier()

    pltpu.sync_copy(acc_sh.at[pl.ds(obase, NR)],
                    xacc_hbm.at[cid].at[pl.ds(loff, NR)])
    pltpu.sync_copy(zbuf, acc_sh.at[pl.ds(obase, NR)])
    cross_sync()

    pltpu.sync_copy(h_hbm.at[pl.ds(nbase, NR)], tl)
    pltpu.sync_copy(acc_sh.at[pl.ds(nbase, NR)], ab)
    pltpu.sync_copy(zbuf, acc_sh.at[pl.ds(nbase, NR)])
    pltpu.sync_copy(xacc_hbm.at[oid].at[pl.ds(loff, NR)], bb)

    @pl.loop(0, NR)
    def _(i):
        deg = ab[i] + bb[i] + 1.0
        dv = _frsqrt(deg)
        t0 = dv * tl[i]
        tl[i] = t0
        gl[i] = ALPHA * t0
        d2l[i] = (1.0 - ALPHA) * dv * dv
        dvl[i] = dv

    pltpu.sync_copy(tl, t_sh.at[pl.ds(nbase, NR)])
    pltpu.sync_copy(tl, xt_hbm.at[cid].at[pl.ds(loff, NR)])
    cross_sync()
    pltpu.sync_copy(xt_hbm.at[oid].at[pl.ds(loff, NR)],
                    t_sh.at[pl.ds(obase, NR)])
    plsc.subcore_barrier()

    def g_start(c, buf, sem):
        pltpu.async_copy(t_sh.at[src_t.at[c]], buf.at[0], sem)

    def g_wait(c, buf, sem):
        pltpu.make_async_copy(t_sh.at[src_t.at[c]], buf.at[0], sem).wait()

    def s_start(c, buf, sem):
        pltpu.async_copy(buf.at[0], acc_sh.at[dst_t.at[c]], sem, add=True)

    def s_wait(c, buf, sem):
        pltpu.make_async_copy(buf.at[0], acc_sh.at[dst_t.at[c]], sem).wait()

    @pl.loop(0, K)
    def _(k):
        g_start(0, msg_a, gs_a)
        g_wait(0, msg_a, gs_a)
        g_start(1, msg_b, gs_b)
        s_start(0, msg_a, ss_a)

        @pl.loop(1, NB // 2)
        def _(p):
            b0 = 2 * p
            b1 = b0 + 1
            s_wait(b0 - 2, msg_a, ss_a)
            g_start(b0, msg_a, gs_a)
            g_wait(b0 - 1, msg_b, gs_b)
            s_start(b0 - 1, msg_b, ss_b)
            s_wait(b0 - 1, msg_b, ss_b)
            g_start(b1, msg_b, gs_b)
            g_wait(b0, msg_a, gs_a)
            s_start(b0, msg_a, ss_a)

        g_wait(NB - 1, msg_b, gs_b)
        s_wait(NB - 2, msg_a, ss_a)
        s_start(NB - 1, msg_b, ss_b)
        s_wait(NB - 1, msg_b, ss_b)
        plsc.subcore_barrier()

        pltpu.sync_copy(acc_sh.at[pl.ds(obase, NR)],
                        xacc_hbm.at[cid].at[pl.ds(loff, NR)])
        pltpu.async_copy(zbuf, acc_sh.at[pl.ds(obase, NR)], ss_b)
        pl.semaphore_signal(xsem, 1, core_index=oid)
        pltpu.make_async_copy(zbuf, acc_sh.at[pl.ds(obase, NR)], ss_b).wait()
        pl.semaphore_wait(xsem, 1)

        pltpu.async_copy(acc_sh.at[pl.ds(nbase, NR)], ab, gs_a)
        pltpu.async_copy(xacc_hbm.at[oid].at[pl.ds(loff, NR)], bb, gs_b)
        pltpu.make_async_copy(acc_sh.at[pl.ds(nbase, NR)], ab, gs_a).wait()
        pltpu.async_copy(zbuf, acc_sh.at[pl.ds(nbase, NR)], ss_a)
        pltpu.make_async_copy(xacc_hbm.at[oid].at[pl.ds(loff, NR)], bb,
                              gs_b).wait()

        @pl.loop(0, NR)
        def _(i):
            tl[i] = d2l[i] * (ab[i] + bb[i] + tl[i]) + gl[i]

        pltpu.async_copy(tl, t_sh.at[pl.ds(nbase, NR)], ss_b)
        pltpu.sync_copy(tl, xt_hbm.at[cid].at[pl.ds(loff, NR)])
        pltpu.make_async_copy(zbuf, acc_sh.at[pl.ds(nbase, NR)], ss_a).wait()
        pltpu.make_async_copy(tl, t_sh.at[pl.ds(nbase, NR)], ss_b).wait()
        cross_sync()
        pltpu.sync_copy(xt_hbm.at[oid].at[pl.ds(loff, NR)],
                        t_sh.at[pl.ds(obase, NR)])
        plsc.subcore_barrier()

    @pl.loop(0, NR)
    def _(i):
        bb[i] = tl[i] / dvl[i]

    pltpu.sync_copy(bb, out_hbm.at[pl.ds(nbase, NR)])


_prop = pl.kernel(
    _prop_body,
    out_type=(
        jax.ShapeDtypeStruct((NP, C), jnp.float32),
        jax.ShapeDtypeStruct((NC, NH, C), jnp.float32),
        jax.ShapeDtypeStruct((NC, NH, C), jnp.float32),
    ),
    mesh=plsc.VectorSubcoreMesh(core_axis_name="c", subcore_axis_name="s",
                                num_cores=NC, num_subcores=NSC),
    compiler_params=pltpu.CompilerParams(use_tc_tiling_on_sc=False),
    scratch_types=[
        pltpu.VMEM_SHARED((NP, C), jnp.float32),
        pltpu.VMEM_SHARED((NP, C), jnp.float32),
        pltpu.VMEM((NCH, CH), jnp.int32),
        pltpu.VMEM((NCH, CH), jnp.int32),
        pltpu.VMEM((1, CH, C), jnp.float32),
        pltpu.VMEM((1, CH, C), jnp.float32),
        pltpu.VMEM((NR, C), jnp.float32),
        pltpu.VMEM((NR, C), jnp.float32),
        pltpu.VMEM((NR, C), jnp.float32),
        pltpu.VMEM((NR, C), jnp.float32),
        pltpu.VMEM((NR, C), jnp.float32),
        pltpu.VMEM((NR, C), jnp.float32),
        pltpu.VMEM((NR, C), jnp.float32),
        pltpu.SemaphoreType.DMA,
        pltpu.SemaphoreType.DMA,
        pltpu.SemaphoreType.DMA,
        pltpu.SemaphoreType.DMA,
        pltpu.SemaphoreType.REGULAR,
    ],
)


@jax.jit
def kernel(x, edge_index, W1, b1, W2, b2):
    h = _mlp(x, W1, b1.reshape(1, H), W2, b2.reshape(1, C))

    npad = EP - edge_index.shape[1]
    rng = jnp.arange(npad, dtype=jnp.int32)
    src = jnp.concatenate([edge_index[0], rng % 64])
    dst = jnp.concatenate([edge_index[1], N + (rng % NJUNK)])
    src_r = src.reshape(NWK, NCH, CH)
    dst_r = dst.reshape(NWK, NCH, CH)

    h_pad = jnp.concatenate([h, jnp.zeros((NP - N, C), jnp.float32)])
    return _prop(h_pad, src_r, dst_r)[0][:N]
